# baseline (speedup 1.0000x reference)
import jax
import jax.numpy as jnp
from jax.experimental import pallas as pl

_N = 10000
_NHID = 64
_NFEAT = 128
_BETA = 0.001
_BM = 400
_GRID = _N // _BM
_BM4 = 400
_GRID4 = _N // _BM4
_BM23 = 1000
_GRID23 = _N // _BM23

_HIGHEST = jax.lax.Precision.HIGHEST


def _dot(a, b):
    return jnp.dot(a, b, preferred_element_type=jnp.float32,
                   precision=_HIGHEST)


def _dot16(a, b):
    return jnp.dot(a.astype(jnp.bfloat16), b.astype(jnp.bfloat16),
                   preferred_element_type=jnp.float32)


def _gc_epilogue(h_in, agg, b):
    out = _BETA * h_in + (1.0 - _BETA) * agg
    nrm = jnp.sqrt(jnp.sum(out * out, axis=1, keepdims=True))
    out = out / jnp.maximum(nrm, 1e-12) + b
    return jnp.maximum(out, 0.0)


def _prologue_kernel(x_ref, wl_ref, bl_ref, wg_ref, h0_ref, sup1_ref):
    h0 = _dot16(x_ref[...], wl_ref[...]) + bl_ref[...]
    h0_ref[...] = h0
    sup1_ref[...] = _dot16(h0, wg_ref[...])


def _pass_kernel(adj_ref, hin_ref, sup_ref, b_ref, wn_ref, hout_ref, supn_ref):
    agg = _dot16(adj_ref[...], sup_ref[...])
    h = _gc_epilogue(hin_ref[...], agg, b_ref[...])
    hout_ref[...] = h
    supn_ref[...] = _dot16(h, wn_ref[...])


def _pass1_kernel(adj_ref, hin_ref, sup_ref, b_ref, wn_ref,
                  hout_ref, supn_ref, adj16_ref):
    adj16 = adj_ref[...].astype(jnp.bfloat16)
    adj16_ref[...] = adj16
    agg = jnp.dot(adj16, sup_ref[...].astype(jnp.bfloat16),
                  preferred_element_type=jnp.float32)
    h = _gc_epilogue(hin_ref[...], agg, b_ref[...])
    hout_ref[...] = h
    supn_ref[...] = _dot16(h, wn_ref[...])


def _pass3_kernel(adj_ref, hin_ref, sup_ref, ba_ref, bs_ref, wn_ref,
                  a1_ref, s_ref, supn_ref):
    agg = _dot16(adj_ref[...], sup_ref[...])
    h_in = hin_ref[...]
    a1 = _gc_epilogue(h_in, agg[:, :_NHID], ba_ref[...])
    s = _gc_epilogue(h_in, agg[:, _NHID:], bs_ref[...])
    a1_ref[...] = a1
    s_ref[...] = s
    supn_ref[...] = _dot16(a1, wn_ref[...])


def _pass4_kernel(adj_ref, hin_ref, sup_ref, b_ref, wlin_ref, blin_ref,
                  sblk_ref, st_ref, xhat_ref, ahat_ref):
    agg = _dot16(adj_ref[...], sup_ref[...])
    a2 = _gc_epilogue(hin_ref[...], agg, b_ref[...])
    xhat_ref[...] = _dot16(a2, wlin_ref[...]) + blin_ref[...]
    ahat_ref[...] = _dot16(sblk_ref[...], st_ref[...])


def _full(shape):
    return pl.BlockSpec(shape, lambda i: (0,) * len(shape))


def _rows(width, bm=_BM):
    return pl.BlockSpec((bm, width), lambda i: (i, 0))


def kernel(x, edge_index, W_enc_lin, b_enc_lin, W_enc_gc1, b_enc_gc1,
           W_enc_gc2, b_enc_gc2, W_att_gc1, b_att_gc1, W_att_gc2, b_att_gc2,
           W_att_lin, b_att_lin, W_str_gc1, b_str_gc1):
    adj = edge_index
    f32 = jnp.float32
    b_enc_lin2 = b_enc_lin.reshape(1, _NHID)
    b_enc_gc1_2 = b_enc_gc1.reshape(1, _NHID)
    b_enc_gc2_2 = b_enc_gc2.reshape(1, _NHID)
    b_att_gc1_2 = b_att_gc1.reshape(1, _NHID)
    b_att_gc2_2 = b_att_gc2.reshape(1, _NHID)
    b_str_gc1_2 = b_str_gc1.reshape(1, _NHID)
    b_att_lin2 = b_att_lin.reshape(1, _NFEAT)
    W_cat = jnp.concatenate([W_att_gc1, W_str_gc1], axis=1)

    h0, sup1 = pl.pallas_call(
        _prologue_kernel,
        out_shape=(jax.ShapeDtypeStruct((_N, _NHID), f32),
                   jax.ShapeDtypeStruct((_N, _NHID), f32)),
    )(x, W_enc_lin, b_enc_lin2, W_enc_gc1)

    h1, sup2, adj16 = pl.pallas_call(
        _pass1_kernel,
        grid=(_GRID,),
        in_specs=[_rows(_N), _rows(_NHID), _full((_N, _NHID)),
                  _full((1, _NHID)), _full((_NHID, _NHID))],
        out_specs=(_rows(_NHID), _rows(_NHID), _rows(_N)),
        out_shape=(jax.ShapeDtypeStruct((_N, _NHID), f32),
                   jax.ShapeDtypeStruct((_N, _NHID), f32),
                   jax.ShapeDtypeStruct((_N, _N), jnp.bfloat16)),
    )(adj, h0, sup1, b_enc_gc1_2, W_enc_gc2)

    h2, sup_cat = pl.pallas_call(
        _pass_kernel,
        grid=(_GRID23,),
        in_specs=[_rows(_N, _BM23), _rows(_NHID, _BM23), _full((_N, _NHID)),
                  _full((1, _NHID)), _full((_NHID, 2 * _NHID))],
        out_specs=(_rows(_NHID, _BM23), _rows(2 * _NHID, _BM23)),
        out_shape=(jax.ShapeDtypeStruct((_N, _NHID), f32),
                   jax.ShapeDtypeStruct((_N, 2 * _NHID), f32)),
    )(adj16, h1, sup2, b_enc_gc2_2, W_cat)

    a1, s, sup_a2 = pl.pallas_call(
        _pass3_kernel,
        grid=(_GRID23,),
        in_specs=[_rows(_N, _BM23), _rows(_NHID, _BM23),
                  _full((_N, 2 * _NHID)),
                  _full((1, _NHID)), _full((1, _NHID)),
                  _full((_NHID, _NHID))],
        out_specs=(_rows(_NHID, _BM23), _rows(_NHID, _BM23),
                   _rows(_NHID, _BM23)),
        out_shape=(jax.ShapeDtypeStruct((_N, _NHID), f32),
                   jax.ShapeDtypeStruct((_N, _NHID), f32),
                   jax.ShapeDtypeStruct((_N, _NHID), f32)),
    )(adj16, h2, sup_cat, b_att_gc1_2, b_str_gc1_2, W_att_gc2)

    s_t = s.T
    x_hat, A_hat = pl.pallas_call(
        _pass4_kernel,
        grid=(_GRID4,),
        in_specs=[_rows(_N, _BM4), _rows(_NHID, _BM4), _full((_N, _NHID)),
                  _full((1, _NHID)), _full((_NHID, _NFEAT)),
                  _full((1, _NFEAT)), _rows(_NHID, _BM4), _full((_NHID, _N))],
        out_specs=(_rows(_NFEAT, _BM4), _rows(_N, _BM4)),
        out_shape=(jax.ShapeDtypeStruct((_N, _NFEAT), f32),
                   jax.ShapeDtypeStruct((_N, _N), f32)),
    )(adj16, a1, sup_a2, b_att_gc2_2, W_att_lin, b_att_lin2, s, s_t)

    return (A_hat, x_hat)


# speedup vs baseline: 1.0546x; 1.0546x over previous
import jax
import jax.numpy as jnp
from jax.experimental import pallas as pl

_N = 10000
_NHID = 64
_NFEAT = 128
_BETA = 0.001
_BM = 400
_GRID = _N // _BM
_BM4 = 200
_GRID4 = _N // _BM4
_BM23 = 1000
_GRID23 = _N // _BM23

_HIGHEST = jax.lax.Precision.HIGHEST


def _dot(a, b):
    return jnp.dot(a, b, preferred_element_type=jnp.float32,
                   precision=_HIGHEST)


def _dot16(a, b):
    return jnp.dot(a.astype(jnp.bfloat16), b.astype(jnp.bfloat16),
                   preferred_element_type=jnp.float32)


def _gc_epilogue(h_in, agg, b):
    out = _BETA * h_in + (1.0 - _BETA) * agg
    nrm = jnp.sqrt(jnp.sum(out * out, axis=1, keepdims=True))
    out = out / jnp.maximum(nrm, 1e-12) + b
    return jnp.maximum(out, 0.0)


def _prologue_kernel(x_ref, wl_ref, bl_ref, wg_ref, h0_ref, sup1_ref):
    h0 = _dot16(x_ref[...], wl_ref[...]) + bl_ref[...]
    h0_ref[...] = h0
    sup1_ref[...] = _dot16(h0, wg_ref[...])


def _pass_kernel(adj_ref, hin_ref, sup_ref, b_ref, wn_ref, hout_ref, supn_ref):
    agg = _dot16(adj_ref[...], sup_ref[...])
    h = _gc_epilogue(hin_ref[...], agg, b_ref[...])
    hout_ref[...] = h
    supn_ref[...] = _dot16(h, wn_ref[...])


def _pass1_kernel(adj_ref, hin_ref, sup_ref, b_ref, wn_ref,
                  hout_ref, supn_ref, adj16_ref):
    adj16 = adj_ref[...].astype(jnp.bfloat16)
    adj16_ref[...] = adj16
    agg = jnp.dot(adj16, sup_ref[...].astype(jnp.bfloat16),
                  preferred_element_type=jnp.float32)
    h = _gc_epilogue(hin_ref[...], agg, b_ref[...])
    hout_ref[...] = h
    supn_ref[...] = _dot16(h, wn_ref[...])


def _pass3_kernel(adj_ref, hin_ref, sup_ref, ba_ref, bs_ref, wn_ref,
                  a1_ref, s_ref, supn_ref):
    agg = _dot16(adj_ref[...], sup_ref[...])
    h_in = hin_ref[...]
    a1 = _gc_epilogue(h_in, agg[:, :_NHID], ba_ref[...])
    s = _gc_epilogue(h_in, agg[:, _NHID:], bs_ref[...])
    a1_ref[...] = a1
    s_ref[...] = s
    supn_ref[...] = _dot16(a1, wn_ref[...])


def _pass4_kernel(adj_ref, hin_ref, sup_ref, b_ref, wlin_ref, blin_ref,
                  sblk_ref, st_ref, xhat_ref, ahat_ref):
    agg = _dot16(adj_ref[...], sup_ref[...])
    a2 = _gc_epilogue(hin_ref[...], agg, b_ref[...])
    xhat_ref[...] = _dot16(a2, wlin_ref[...]) + blin_ref[...]
    ahat_ref[...] = _dot16(sblk_ref[...], st_ref[...])


def _full(shape):
    return pl.BlockSpec(shape, lambda i: (0,) * len(shape))


def _rows(width, bm=_BM):
    return pl.BlockSpec((bm, width), lambda i: (i, 0))


def kernel(x, edge_index, W_enc_lin, b_enc_lin, W_enc_gc1, b_enc_gc1,
           W_enc_gc2, b_enc_gc2, W_att_gc1, b_att_gc1, W_att_gc2, b_att_gc2,
           W_att_lin, b_att_lin, W_str_gc1, b_str_gc1):
    adj = edge_index
    f32 = jnp.float32
    b_enc_lin2 = b_enc_lin.reshape(1, _NHID)
    b_enc_gc1_2 = b_enc_gc1.reshape(1, _NHID)
    b_enc_gc2_2 = b_enc_gc2.reshape(1, _NHID)
    b_att_gc1_2 = b_att_gc1.reshape(1, _NHID)
    b_att_gc2_2 = b_att_gc2.reshape(1, _NHID)
    b_str_gc1_2 = b_str_gc1.reshape(1, _NHID)
    b_att_lin2 = b_att_lin.reshape(1, _NFEAT)
    W_cat = jnp.concatenate([W_att_gc1, W_str_gc1], axis=1)

    h0, sup1 = pl.pallas_call(
        _prologue_kernel,
        out_shape=(jax.ShapeDtypeStruct((_N, _NHID), f32),
                   jax.ShapeDtypeStruct((_N, _NHID), f32)),
    )(x, W_enc_lin, b_enc_lin2, W_enc_gc1)

    h1, sup2, adj16 = pl.pallas_call(
        _pass1_kernel,
        grid=(_GRID,),
        in_specs=[_rows(_N), _rows(_NHID), _full((_N, _NHID)),
                  _full((1, _NHID)), _full((_NHID, _NHID))],
        out_specs=(_rows(_NHID), _rows(_NHID), _rows(_N)),
        out_shape=(jax.ShapeDtypeStruct((_N, _NHID), f32),
                   jax.ShapeDtypeStruct((_N, _NHID), f32),
                   jax.ShapeDtypeStruct((_N, _N), jnp.bfloat16)),
    )(adj, h0, sup1, b_enc_gc1_2, W_enc_gc2)

    h2, sup_cat = pl.pallas_call(
        _pass_kernel,
        grid=(_GRID23,),
        in_specs=[_rows(_N, _BM23), _rows(_NHID, _BM23), _full((_N, _NHID)),
                  _full((1, _NHID)), _full((_NHID, 2 * _NHID))],
        out_specs=(_rows(_NHID, _BM23), _rows(2 * _NHID, _BM23)),
        out_shape=(jax.ShapeDtypeStruct((_N, _NHID), f32),
                   jax.ShapeDtypeStruct((_N, 2 * _NHID), f32)),
    )(adj16, h1, sup2, b_enc_gc2_2, W_cat)

    a1, s, sup_a2 = pl.pallas_call(
        _pass3_kernel,
        grid=(_GRID23,),
        in_specs=[_rows(_N, _BM23), _rows(_NHID, _BM23),
                  _full((_N, 2 * _NHID)),
                  _full((1, _NHID)), _full((1, _NHID)),
                  _full((_NHID, _NHID))],
        out_specs=(_rows(_NHID, _BM23), _rows(_NHID, _BM23),
                   _rows(_NHID, _BM23)),
        out_shape=(jax.ShapeDtypeStruct((_N, _NHID), f32),
                   jax.ShapeDtypeStruct((_N, _NHID), f32),
                   jax.ShapeDtypeStruct((_N, _NHID), f32)),
    )(adj16, h2, sup_cat, b_att_gc1_2, b_str_gc1_2, W_att_gc2)

    s_t = s.T
    x_hat, A_hat = pl.pallas_call(
        _pass4_kernel,
        grid=(_GRID4,),
        in_specs=[_rows(_N, _BM4), _rows(_NHID, _BM4), _full((_N, _NHID)),
                  _full((1, _NHID)), _full((_NHID, _NFEAT)),
                  _full((1, _NFEAT)), _rows(_NHID, _BM4), _full((_NHID, _N))],
        out_specs=(_rows(_NFEAT, _BM4), _rows(_N, _BM4)),
        out_shape=(jax.ShapeDtypeStruct((_N, _NFEAT), f32),
                   jax.ShapeDtypeStruct((_N, _N), f32)),
    )(adj16, a1, sup_a2, b_att_gc2_2, W_att_lin, b_att_lin2, s, s_t)

    return (A_hat, x_hat)
